# baseline (speedup 1.0000x reference)
import math

import functools

import jax
import jax.numpy as jnp
from jax import lax
from jax.experimental import pallas as pl
from jax.experimental.pallas import tpu as pltpu
from jax.experimental.pallas import tpu_sc as plsc

_NUM_NODE_TYPE = 4
_NODE_TYPE_EMB = 128
_HIDDEN = 1024
_NUM_LAYERS = 4
_LAMBDA = 0.5
_ALPHA = 0.1
_N = 1024
_DB = 1028

_BETAS = tuple(float(math.log(_LAMBDA / (l + 1) + 1.0)) for l in range(_NUM_LAYERS))


_E = 65536
_NC = 2
_NS = 16
_EPW = _E // (_NC * _NS)
_SLICE = _N * _N // _NS


def _edge_count_sc_kernel(edge_hbm, out_hbm,
                          src_v, dst_v, idx_v, ones_v, zero_v, c_shared):
    cid = lax.axis_index("c")
    sid = lax.axis_index("s")

    for i in range(8):
        ones_v[pl.ds(i * 16, 16)] = jnp.ones((16,), jnp.float32)

    def _z(i, _):
        zero_v[pl.ds(i * 16, 16)] = jnp.zeros((16,), jnp.float32)
        return _
    lax.fori_loop(0, zero_v.shape[0] // 16, _z, 0)

    zlen = zero_v.shape[0]
    for i in range(_SLICE // zlen):
        pltpu.sync_copy(zero_v, c_shared.at[pl.ds(sid * _SLICE + i * zlen, zlen)])

    base = cid * (_NS * _EPW) + sid * _EPW
    pltpu.sync_copy(edge_hbm.at[0, pl.ds(base, _EPW)], src_v)
    pltpu.sync_copy(edge_hbm.at[1, pl.ds(base, _EPW)], dst_v)

    for j in range(_EPW // 128):
        for i in range(8):
            p = j * 128 + i * 16
            idx_v[j, pl.ds(i * 16, 16)] = (
                dst_v[pl.ds(p, 16)] * _N + src_v[pl.ds(p, 16)])

    plsc.subcore_barrier()

    for j in range(_EPW // 128):
        pltpu.sync_copy(ones_v, c_shared.at[idx_v.at[j]], add=True)

    plsc.subcore_barrier()

    pltpu.sync_copy(c_shared.at[pl.ds(sid * _SLICE, _SLICE)],
                    out_hbm.at[pl.ds(cid * (_N * _N) + sid * _SLICE, _SLICE)])


def _edge_counts(edge_index):
    k = functools.partial(
        pl.kernel,
        out_type=jax.ShapeDtypeStruct((_NC * _N * _N,), jnp.float32),
        mesh=plsc.VectorSubcoreMesh(core_axis_name="c", subcore_axis_name="s"),
        scratch_types=[
            pltpu.VMEM((_EPW,), jnp.int32),
            pltpu.VMEM((_EPW,), jnp.int32),
            pltpu.VMEM((_EPW // 128, 128), jnp.int32),
            pltpu.VMEM((128,), jnp.float32),
            pltpu.VMEM((4096,), jnp.float32),
            pltpu.VMEM_SHARED((_N * _N,), jnp.float32),
        ],
    )(_edge_count_sc_kernel)
    return k(edge_index).reshape(_NC, _N, _N)


_TB = 256


def _build_a_kernel(c2_ref, a_ref):
    c = c2_ref[0] + c2_ref[1]
    deg = jnp.sum(c, axis=1, keepdims=True) + 1.0
    r = jax.lax.rsqrt(deg)
    eye = (jax.lax.broadcasted_iota(jnp.int32, (_N, _N), 0)
           == jax.lax.broadcasted_iota(jnp.int32, (_N, _N), 1)
           ).astype(jnp.float32)
    r_row = jnp.dot(jnp.ones((8, _N), jnp.float32), eye * r,
                    preferred_element_type=jnp.float32,
                    precision=jax.lax.Precision.HIGHEST)[:1]
    a_ref[...] = (c + eye) * r * r_row


def _gcnii_tc_kernel(a_ref, h0a_ref, h0b_ref, wa_ref, wb_ref,
                     out_ref, ha_ref, hb_ref, t_ref):
    l = pl.program_id(0)

    @pl.when(l == 0)
    def _init():
        ha_ref[...] = h0a_ref[...]
        hb_ref[...] = h0b_ref[...]

    beta = jnp.where(l == 0, _BETAS[0],
                     jnp.where(l == 1, _BETAS[1],
                               jnp.where(l == 2, _BETAS[2], _BETAS[3])))
    beta = beta.astype(jnp.float32)

    def layer(h_ref, h0_ref, w_ref, width, out_col):
        for i in range(_N // _TB):
            rows = pl.ds(i * _TB, _TB)
            agg = jnp.dot(a_ref[rows, :], h_ref[...],
                          preferred_element_type=jnp.float32,
                          precision=jax.lax.Precision.DEFAULT)
            t_ref[rows, :width] = (1.0 - _ALPHA) * agg + _ALPHA * h0_ref[rows, :]
        for i in range(_N // _TB):
            rows = pl.ds(i * _TB, _TB)
            t = t_ref[rows, :width]
            tw = jnp.dot(t, w_ref[0], preferred_element_type=jnp.float32,
                         precision=jax.lax.Precision.DEFAULT)
            hn = jax.nn.relu((1.0 - beta) * t + beta * tw)

            @pl.when(l < _NUM_LAYERS - 1)
            def _carry():
                h_ref[rows, :] = hn

            @pl.when(l == _NUM_LAYERS - 1)
            def _emit():
                out_ref[rows, pl.ds(out_col, width)] = hn

    layer(ha_ref, h0a_ref, wa_ref, _HIDDEN, _DB)
    layer(hb_ref, h0b_ref, wb_ref, _DB, 0)


def _run_gcnii(c2, h0a, h0b, wa, wb):
    a_norm = pl.pallas_call(
        _build_a_kernel,
        out_shape=jax.ShapeDtypeStruct((_N, _N), jnp.float32),
    )(c2)
    full = lambda shape: pl.BlockSpec(shape, lambda l: (0,) * len(shape))
    per_layer = lambda shape: pl.BlockSpec((1,) + shape, lambda l: (l, 0, 0))
    out = pl.pallas_call(
        _gcnii_tc_kernel,
        grid=(_NUM_LAYERS,),
        in_specs=[
            full((_N, _N)),
            full((_N, _HIDDEN)),
            full((_N, _DB)),
            per_layer((_HIDDEN, _HIDDEN)),
            per_layer((_DB, _DB)),
        ],
        out_specs=full((_N, _DB + _HIDDEN)),
        out_shape=jax.ShapeDtypeStruct((_N, _DB + _HIDDEN), jnp.float32),
        scratch_shapes=[
            pltpu.VMEM((_N, _HIDDEN), jnp.float32),
            pltpu.VMEM((_N, _DB), jnp.float32),
            pltpu.VMEM((_N, _DB), jnp.float32),
        ],
    )(a_norm, h0a, h0b, wa, wb)
    return out


def kernel(one_hot_encoding, mention_hidden_state, entity_hidden_state,
           sent_hidden_state, token_hidden_state, edge_index,
           node_type_embedding, gcn_W, gcn_oh_W):
    num_node = one_hot_encoding.shape[0]
    batch_size, num_mention, _ = mention_hidden_state.shape
    num_entity = int(entity_hidden_state.shape[1])
    num_sent = int(sent_hidden_state.shape[1])
    num_token = int(token_hidden_state.shape[1])

    def tile(emb, n):
        return jnp.broadcast_to(emb.reshape(1, 1, -1), (batch_size, n, emb.shape[-1]))

    m = jnp.concatenate([mention_hidden_state, tile(node_type_embedding[0], num_mention)], axis=2)
    e = jnp.concatenate([entity_hidden_state, tile(node_type_embedding[1], num_entity)], axis=2)
    s = jnp.concatenate([sent_hidden_state, tile(node_type_embedding[2], num_sent)], axis=2)
    t = jnp.concatenate([token_hidden_state, tile(node_type_embedding[3], num_token)], axis=2)
    node_h = jnp.concatenate([m, e, s, t], axis=1)
    n_per = int(node_h.shape[1])
    h0a = node_h.reshape(batch_size * n_per, -1)

    d2 = _HIDDEN + _NUM_NODE_TYPE
    flags = jnp.zeros((num_node, _NUM_NODE_TYPE), jnp.float32)
    flags = flags.at[:num_mention, 0].set(1.0)
    flags = flags.at[num_mention:num_mention + num_entity, 1].set(1.0)
    flags = flags.at[num_mention + num_entity:num_mention + num_entity + num_sent, 2].set(1.0)
    flags = flags.at[num_mention + num_entity + num_sent:
                     num_mention + num_entity + num_sent + num_token, 3].set(1.0)
    h0b = jnp.concatenate([flags, one_hot_encoding], axis=1)

    c2 = _edge_counts(edge_index)

    out = _run_gcnii(c2, h0a, h0b, gcn_W, gcn_oh_W)
    out3 = out.reshape(batch_size, n_per, -1)
    entity_out = out3[:, num_mention:num_mention + num_entity]
    out_flat = out3.reshape(batch_size * n_per, -1)
    return (entity_out, out_flat)


# speedup vs baseline: 1.0764x; 1.0764x over previous
import math

import functools

import jax
import jax.numpy as jnp
from jax import lax
from jax.experimental import pallas as pl
from jax.experimental.pallas import tpu as pltpu
from jax.experimental.pallas import tpu_sc as plsc

_NUM_NODE_TYPE = 4
_NODE_TYPE_EMB = 128
_HIDDEN = 1024
_NUM_LAYERS = 4
_LAMBDA = 0.5
_ALPHA = 0.1
_N = 1024
_DB = 1028

_BETAS = tuple(float(math.log(_LAMBDA / (l + 1) + 1.0)) for l in range(_NUM_LAYERS))


_E = 65536
_NC = 2
_NS = 16
_EPW = _E // (_NC * _NS)
_SLICE = _N * _N // _NS


def _edge_count_sc_kernel(edge_hbm, out_hbm,
                          src_v, dst_v, idx_v, ones_v, zero_v, c_shared):
    cid = lax.axis_index("c")
    sid = lax.axis_index("s")

    for i in range(8):
        ones_v[pl.ds(i * 16, 16)] = jnp.ones((16,), jnp.float32)

    def _z(i, _):
        zero_v[pl.ds(i * 16, 16)] = jnp.zeros((16,), jnp.float32)
        return _
    lax.fori_loop(0, zero_v.shape[0] // 16, _z, 0)

    zlen = zero_v.shape[0]
    for i in range(_SLICE // zlen):
        pltpu.sync_copy(zero_v, c_shared.at[pl.ds(sid * _SLICE + i * zlen, zlen)])

    base = cid * (_NS * _EPW) + sid * _EPW
    pltpu.sync_copy(edge_hbm.at[0, pl.ds(base, _EPW)], src_v)
    pltpu.sync_copy(edge_hbm.at[1, pl.ds(base, _EPW)], dst_v)

    for j in range(_EPW // 128):
        for i in range(8):
            p = j * 128 + i * 16
            idx_v[j, pl.ds(i * 16, 16)] = (
                dst_v[pl.ds(p, 16)] * _N + src_v[pl.ds(p, 16)])

    plsc.subcore_barrier()

    for j in range(_EPW // 128):
        pltpu.sync_copy(ones_v, c_shared.at[idx_v.at[j]], add=True)

    plsc.subcore_barrier()

    pltpu.sync_copy(c_shared.at[pl.ds(sid * _SLICE, _SLICE)],
                    out_hbm.at[pl.ds(cid * (_N * _N) + sid * _SLICE, _SLICE)])


def _edge_counts(edge_index):
    k = functools.partial(
        pl.kernel,
        out_type=jax.ShapeDtypeStruct((_NC * _N * _N,), jnp.float32),
        mesh=plsc.VectorSubcoreMesh(core_axis_name="c", subcore_axis_name="s"),
        scratch_types=[
            pltpu.VMEM((_EPW,), jnp.int32),
            pltpu.VMEM((_EPW,), jnp.int32),
            pltpu.VMEM((_EPW // 128, 128), jnp.int32),
            pltpu.VMEM((128,), jnp.float32),
            pltpu.VMEM((4096,), jnp.float32),
            pltpu.VMEM_SHARED((_N * _N,), jnp.float32),
        ],
    )(_edge_count_sc_kernel)
    return k(edge_index).reshape(_NC, _N, _N)


_TB = 256


def _gcnii_tc_kernel(c2_ref, h0a_ref, h0b_ref, wa_ref, wb_ref,
                     outa_ref, outb_ref, a_ref, t_ref):
    l = pl.program_id(0)

    @pl.when(l == 0)
    def _init():
        c = c2_ref[0] + c2_ref[1]
        deg = jnp.sum(c, axis=1, keepdims=True) + 1.0
        r = jax.lax.rsqrt(deg)
        eye = (jax.lax.broadcasted_iota(jnp.int32, (_N, _N), 0)
               == jax.lax.broadcasted_iota(jnp.int32, (_N, _N), 1)
               ).astype(jnp.float32)
        r_row = jnp.dot(jnp.ones((8, _N), jnp.float32), eye * r,
                        preferred_element_type=jnp.float32,
                        precision=jax.lax.Precision.HIGHEST)[:1]
        a_ref[...] = (c + eye) * r * r_row
        outa_ref[...] = h0a_ref[...]
        outb_ref[...] = h0b_ref[...]

    beta = jnp.where(l == 0, _BETAS[0],
                     jnp.where(l == 1, _BETAS[1],
                               jnp.where(l == 2, _BETAS[2], _BETAS[3])))
    beta = beta.astype(jnp.float32)

    def layer(h_ref, h0_ref, w_ref, width):
        for i in range(_N // _TB):
            rows = pl.ds(i * _TB, _TB)
            agg = jnp.dot(a_ref[rows, :], h_ref[...],
                          preferred_element_type=jnp.float32,
                          precision=jax.lax.Precision.DEFAULT)
            t_ref[rows, :width] = (1.0 - _ALPHA) * agg + _ALPHA * h0_ref[rows, :]
        for i in range(_N // _TB):
            rows = pl.ds(i * _TB, _TB)
            t = t_ref[rows, :width]
            tw = jnp.dot(t, w_ref[0], preferred_element_type=jnp.float32,
                         precision=jax.lax.Precision.DEFAULT)
            h_ref[rows, :] = jax.nn.relu((1.0 - beta) * t + beta * tw)

    layer(outa_ref, h0a_ref, wa_ref, _HIDDEN)
    layer(outb_ref, h0b_ref, wb_ref, _DB)


def _run_gcnii(c2, h0a, h0b, wa, wb):
    full = lambda shape: pl.BlockSpec(shape, lambda l: (0,) * len(shape))
    per_layer = lambda shape: pl.BlockSpec((1,) + shape, lambda l: (l, 0, 0))
    outa, outb = pl.pallas_call(
        _gcnii_tc_kernel,
        grid=(_NUM_LAYERS,),
        in_specs=[
            full((2, _N, _N)),
            full((_N, _HIDDEN)),
            full((_N, _DB)),
            per_layer((_HIDDEN, _HIDDEN)),
            per_layer((_DB, _DB)),
        ],
        out_specs=[full((_N, _HIDDEN)), full((_N, _DB))],
        out_shape=[
            jax.ShapeDtypeStruct((_N, _HIDDEN), jnp.float32),
            jax.ShapeDtypeStruct((_N, _DB), jnp.float32),
        ],
        scratch_shapes=[
            pltpu.VMEM((_N, _N), jnp.float32),
            pltpu.VMEM((_N, _DB), jnp.float32),
        ],
    )(c2, h0a, h0b, wa, wb)
    return outa, outb


def kernel(one_hot_encoding, mention_hidden_state, entity_hidden_state,
           sent_hidden_state, token_hidden_state, edge_index,
           node_type_embedding, gcn_W, gcn_oh_W):
    num_node = one_hot_encoding.shape[0]
    batch_size, num_mention, _ = mention_hidden_state.shape
    num_entity = int(entity_hidden_state.shape[1])
    num_sent = int(sent_hidden_state.shape[1])
    num_token = int(token_hidden_state.shape[1])

    def tile(emb, n):
        return jnp.broadcast_to(emb.reshape(1, 1, -1), (batch_size, n, emb.shape[-1]))

    m = jnp.concatenate([mention_hidden_state, tile(node_type_embedding[0], num_mention)], axis=2)
    e = jnp.concatenate([entity_hidden_state, tile(node_type_embedding[1], num_entity)], axis=2)
    s = jnp.concatenate([sent_hidden_state, tile(node_type_embedding[2], num_sent)], axis=2)
    t = jnp.concatenate([token_hidden_state, tile(node_type_embedding[3], num_token)], axis=2)
    node_h = jnp.concatenate([m, e, s, t], axis=1)
    n_per = int(node_h.shape[1])
    h0a = node_h.reshape(batch_size * n_per, -1)

    d2 = _HIDDEN + _NUM_NODE_TYPE
    flags = jnp.zeros((num_node, _NUM_NODE_TYPE), jnp.float32)
    flags = flags.at[:num_mention, 0].set(1.0)
    flags = flags.at[num_mention:num_mention + num_entity, 1].set(1.0)
    flags = flags.at[num_mention + num_entity:num_mention + num_entity + num_sent, 2].set(1.0)
    flags = flags.at[num_mention + num_entity + num_sent:
                     num_mention + num_entity + num_sent + num_token, 3].set(1.0)
    h0b = jnp.concatenate([flags, one_hot_encoding], axis=1)

    c2 = _edge_counts(edge_index)

    outa, outb = _run_gcnii(c2, h0a, h0b, gcn_W, gcn_oh_W)

    out = jnp.concatenate([outb, outa], axis=1)
    out3 = out.reshape(batch_size, n_per, -1)
    entity_out = out3[:, num_mention:num_mention + num_entity]
    out_flat = out3.reshape(batch_size * n_per, -1)
    return (entity_out, out_flat)


# speedup vs baseline: 1.1301x; 1.0499x over previous
import functools
import math

import jax
import jax.numpy as jnp
from jax import lax
from jax.experimental import pallas as pl
from jax.experimental.pallas import tpu as pltpu
from jax.experimental.pallas import tpu_sc as plsc

_NUM_NODE_TYPE = 4
_NODE_TYPE_EMB = 128
_HIDDEN = 1024
_DIN = _HIDDEN - _NODE_TYPE_EMB
_NUM_LAYERS = 4
_LAMBDA = 0.5
_ALPHA = 0.1
_N = 1024
_DB = 1028
_B = 4
_NM, _NE, _NS_SENT, _NT = 80, 40, 24, 112
_NPER = _NM + _NE + _NS_SENT + _NT

_BETAS = tuple(float(math.log(_LAMBDA / (l + 1) + 1.0)) for l in range(_NUM_LAYERS))

_E = 65536
_NC = 2
_NS = 16
_EPW = _E // (_NC * _NS)
_SLICE = _N * _N // _NS


def _edge_count_sc_kernel(edge_hbm, out_hbm,
                          src_v, dst_v, idx_v, ones_v, zero_v, c_shared):
    cid = lax.axis_index("c")
    sid = lax.axis_index("s")

    for i in range(8):
        ones_v[pl.ds(i * 16, 16)] = jnp.ones((16,), jnp.float32)

    def _z(i, carry):
        zero_v[pl.ds(i * 16, 16)] = jnp.zeros((16,), jnp.float32)
        return carry
    lax.fori_loop(0, zero_v.shape[0] // 16, _z, 0)

    zlen = zero_v.shape[0]
    for i in range(_SLICE // zlen):
        pltpu.sync_copy(zero_v, c_shared.at[pl.ds(sid * _SLICE + i * zlen, zlen)])

    base = cid * (_NS * _EPW) + sid * _EPW
    pltpu.sync_copy(edge_hbm.at[0, pl.ds(base, _EPW)], src_v)
    pltpu.sync_copy(edge_hbm.at[1, pl.ds(base, _EPW)], dst_v)

    for j in range(_EPW // 128):
        for i in range(8):
            p = j * 128 + i * 16
            idx_v[j, pl.ds(i * 16, 16)] = (
                dst_v[pl.ds(p, 16)] * _N + src_v[pl.ds(p, 16)])

    plsc.subcore_barrier()

    for j in range(_EPW // 128):
        pltpu.sync_copy(ones_v, c_shared.at[idx_v.at[j]], add=True)

    plsc.subcore_barrier()

    pltpu.sync_copy(c_shared.at[pl.ds(sid * _SLICE, _SLICE)],
                    out_hbm.at[pl.ds(cid * (_N * _N) + sid * _SLICE, _SLICE)])


def _edge_counts(edge_index):
    k = functools.partial(
        pl.kernel,
        out_type=jax.ShapeDtypeStruct((_NC * _N * _N,), jnp.float32),
        mesh=plsc.VectorSubcoreMesh(core_axis_name="c", subcore_axis_name="s"),
        scratch_types=[
            pltpu.VMEM((_EPW,), jnp.int32),
            pltpu.VMEM((_EPW,), jnp.int32),
            pltpu.VMEM((_EPW // 128, 128), jnp.int32),
            pltpu.VMEM((128,), jnp.float32),
            pltpu.VMEM((4096,), jnp.float32),
            pltpu.VMEM_SHARED((_N * _N,), jnp.float32),
        ],
    )(_edge_count_sc_kernel)
    return k(edge_index).reshape(_NC, _N, _N)


_TB = 256


def _gcnii_tc_kernel(c2_ref, m_ref, e_ref, s_ref, tk_ref, emb_ref, oh_ref,
                     wa_ref, wb_ref, outa_ref, outb_ref,
                     a_ref, t_ref, h0a_ref, h0b_ref, sem):
    l = pl.program_id(0)

    @pl.when(l == 0)
    def _init():
        cp0 = pltpu.make_async_copy(c2_ref.at[0], a_ref, sem)
        cp1 = pltpu.make_async_copy(c2_ref.at[1], h0a_ref, sem)
        cp0.start()
        cp1.start()
        cp0.wait()
        cp1.wait()
        c = a_ref[...] + h0a_ref[...]
        deg = jnp.sum(c, axis=1, keepdims=True) + 1.0
        r = jax.lax.rsqrt(deg)
        eye = (jax.lax.broadcasted_iota(jnp.int32, (_N, _N), 0)
               == jax.lax.broadcasted_iota(jnp.int32, (_N, _N), 1)
               ).astype(jnp.float32)
        r_row = jnp.dot(jnp.ones((8, _N), jnp.float32), eye * r,
                        preferred_element_type=jnp.float32,
                        precision=jax.lax.Precision.HIGHEST)[:1]
        a_ref[...] = (c + eye) * r * r_row

        for b in range(_B):
            base = b * _NPER
            for (ref, cnt, off, ty) in ((m_ref, _NM, 0, 0),
                                        (e_ref, _NE, _NM, 1),
                                        (s_ref, _NS_SENT, _NM + _NE, 2),
                                        (tk_ref, _NT, _NM + _NE + _NS_SENT, 3)):
                h0a_ref[pl.ds(base + off, cnt), :_DIN] = ref[b]
                h0a_ref[pl.ds(base + off, cnt), _DIN:] = jnp.broadcast_to(
                    emb_ref[ty:ty + 1, :], (cnt, _NODE_TYPE_EMB))

        row4 = jax.lax.broadcasted_iota(jnp.int32, (_N, _NUM_NODE_TYPE), 0)
        col4 = jax.lax.broadcasted_iota(jnp.int32, (_N, _NUM_NODE_TYPE), 1)
        lo = ((col4 == 1) * _NM + (col4 == 2) * (_NM + _NE)
              + (col4 == 3) * (_NM + _NE + _NS_SENT))
        hi = ((col4 == 0) * _NM + (col4 == 1) * (_NM + _NE)
              + (col4 == 2) * (_NM + _NE + _NS_SENT) + (col4 == 3) * _NPER)
        flag4 = ((row4 >= lo) & (row4 < hi)).astype(jnp.float32)
        h0b_ref[:, :128] = jnp.concatenate(
            [flag4, oh_ref[:, :128 - _NUM_NODE_TYPE]], axis=1)
        h0b_ref[:, pl.ds(128, _N - 128 + _NUM_NODE_TYPE)] = (
            oh_ref[:, pl.ds(128 - _NUM_NODE_TYPE, _N - 128 + _NUM_NODE_TYPE)])

        outa_ref[...] = h0a_ref[...]
        outb_ref[...] = h0b_ref[...]

    beta = jnp.where(l == 0, _BETAS[0],
                     jnp.where(l == 1, _BETAS[1],
                               jnp.where(l == 2, _BETAS[2], _BETAS[3])))
    beta = beta.astype(jnp.float32)

    def layer(h_ref, h0_ref, w_ref, width):
        for i in range(_N // _TB):
            rows = pl.ds(i * _TB, _TB)
            agg = jnp.dot(a_ref[rows, :], h_ref[...],
                          preferred_element_type=jnp.float32,
                          precision=jax.lax.Precision.DEFAULT)
            t_ref[rows, :width] = (1.0 - _ALPHA) * agg + _ALPHA * h0_ref[rows, :width]
        for i in range(_N // _TB):
            rows = pl.ds(i * _TB, _TB)
            t = t_ref[rows, :width]
            tw = jnp.dot(t, w_ref[0], preferred_element_type=jnp.float32,
                         precision=jax.lax.Precision.DEFAULT)
            h_ref[rows, :] = jax.nn.relu((1.0 - beta) * t + beta * tw)

    layer(outa_ref, h0a_ref, wa_ref, _HIDDEN)
    layer(outb_ref, h0b_ref, wb_ref, _DB)


def _run_gcnii(c2, m, e, s, tk, emb8, oh, wa, wb):
    full = lambda shape: pl.BlockSpec(shape, lambda l: (0,) * len(shape))
    per_layer = lambda shape: pl.BlockSpec((1,) + shape, lambda l: (l, 0, 0))
    outa, outb = pl.pallas_call(
        _gcnii_tc_kernel,
        grid=(_NUM_LAYERS,),
        in_specs=[
            pl.BlockSpec(memory_space=pl.ANY),
            full((_B, _NM, _DIN)),
            full((_B, _NE, _DIN)),
            full((_B, _NS_SENT, _DIN)),
            full((_B, _NT, _DIN)),
            full((8, _NODE_TYPE_EMB)),
            full((_N, _N)),
            per_layer((_HIDDEN, _HIDDEN)),
            per_layer((_DB, _DB)),
        ],
        out_specs=[full((_N, _HIDDEN)), full((_N, _DB))],
        out_shape=[
            jax.ShapeDtypeStruct((_N, _HIDDEN), jnp.float32),
            jax.ShapeDtypeStruct((_N, _DB), jnp.float32),
        ],
        scratch_shapes=[
            pltpu.VMEM((_N, _N), jnp.float32),
            pltpu.VMEM((_N, _DB), jnp.float32),
            pltpu.VMEM((_N, _HIDDEN), jnp.float32),
            pltpu.VMEM((_N, _DB), jnp.float32),
            pltpu.SemaphoreType.DMA,
        ],
    )(c2, m, e, s, tk, emb8, oh, wa, wb)
    return outa, outb


def kernel(one_hot_encoding, mention_hidden_state, entity_hidden_state,
           sent_hidden_state, token_hidden_state, edge_index,
           node_type_embedding, gcn_W, gcn_oh_W):
    batch_size, num_mention, _ = mention_hidden_state.shape
    num_entity = int(entity_hidden_state.shape[1])

    c2 = _edge_counts(edge_index)

    emb8 = jnp.pad(node_type_embedding, ((0, 8 - _NUM_NODE_TYPE), (0, 0)))

    outa, outb = _run_gcnii(c2, mention_hidden_state, entity_hidden_state,
                            sent_hidden_state, token_hidden_state, emb8,
                            one_hot_encoding, gcn_W, gcn_oh_W)

    out = jnp.concatenate([outb, outa], axis=1)
    out3 = out.reshape(batch_size, _NPER, -1)
    entity_out = out3[:, num_mention:num_mention + num_entity]
    out_flat = out3.reshape(batch_size * _NPER, -1)
    return (entity_out, out_flat)
